# baseline (speedup 1.0000x reference)
import jax
import jax.numpy as jnp
from jax.experimental import pallas as pl
from jax.experimental.pallas import tpu as pltpu
from jax.experimental.pallas import tpu_sc as plsc

T = 2048
D = 768
E = 8
K = 2
TB = 256
BLK = 256
G = (T * K) // BLK + E
P = G * BLK
A = T * K
R = D // 128
SCW = 128


def _router_kernel(x_ref, wr_ref, widx_ref, w_ref, rank_ref, cnt_ref,
                   aux_ref):
  x = x_ref[...]
  logits = jnp.dot(x, wr_ref[...], preferred_element_type=jnp.float32)
  m = jnp.max(logits, axis=-1, keepdims=True)
  p = jnp.exp(logits - m)
  p = p / jnp.sum(p, axis=-1, keepdims=True)

  lane = jax.lax.broadcasted_iota(jnp.int32, (T, E), 1)
  e0 = jnp.argmax(p, axis=-1).astype(jnp.int32)
  p0 = jnp.max(p, axis=-1)
  oh0 = (lane == e0[:, None]).astype(jnp.float32)
  p_m = jnp.where(oh0 > 0, -jnp.inf, p)
  e1 = jnp.argmax(p_m, axis=-1).astype(jnp.int32)
  p1 = jnp.max(p_m, axis=-1)
  oh1 = (lane == e1[:, None]).astype(jnp.float32)

  s = p0 + p1
  w0 = p0 / s
  w1 = p1 / s

  comb = oh0 * w0[:, None] + oh1 * w1[:, None]
  widx_ref[...] = jnp.concatenate([e0[:, None], e1[:, None]], axis=1)
  w_ref[...] = jnp.concatenate([w0[:, None], w1[:, None]], axis=1)

  cnt2 = oh0 + oh1
  r = jax.lax.broadcasted_iota(jnp.int32, (TB, TB), 0)
  c = jax.lax.broadcasted_iota(jnp.int32, (TB, TB), 1)
  tri = (r > c).astype(jnp.float32)
  carry = jnp.zeros((1, E), jnp.float32)
  ce_blocks = []
  for i in range(T // TB):
    blk = cnt2[i * TB:(i + 1) * TB]
    ce_blocks.append(jnp.dot(tri, blk, preferred_element_type=jnp.float32)
                     + carry)
    carry = carry + jnp.sum(blk, axis=0, keepdims=True)
  ce = jnp.concatenate(ce_blocks, axis=0)

  rank0 = jnp.sum(ce * oh0, axis=-1)
  rank1 = jnp.sum(ce * oh1, axis=-1)
  rank_ref[...] = jnp.concatenate(
      [rank0[:, None], rank1[:, None]], axis=1).astype(jnp.int32)
  cnt_ref[...] = carry.astype(jnp.int32)

  density = carry / T
  importance = jnp.sum(comb, axis=0, keepdims=True) / T
  aux_ref[...] = E * jnp.sum(density * importance, axis=-1, keepdims=True)


def _router(x, wr):
  return pl.pallas_call(
      _router_kernel,
      out_shape=(
          jax.ShapeDtypeStruct((T, K), jnp.int32),
          jax.ShapeDtypeStruct((T, K), jnp.float32),
          jax.ShapeDtypeStruct((T, K), jnp.int32),
          jax.ShapeDtypeStruct((1, E), jnp.int32),
          jax.ShapeDtypeStruct((1, 1), jnp.float32),
      ),
  )(x, wr)


def _vector_mesh():
  return plsc.VectorSubcoreMesh(core_axis_name="c", subcore_axis_name="s")


def _sc_dispatch(x, pos6):
  x6 = x.reshape(T * R, 128)

  @pl.kernel(out_type=jax.ShapeDtypeStruct((P * R, 128), jnp.float32),
             mesh=_vector_mesh())
  def k(x_hbm, i_hbm, o_hbm):
    def body(x_vmem, i_vmem):
      pltpu.sync_copy(x_vmem, o_hbm.at[i_vmem.at[0]])

    pltpu.emit_pipeline(
        body,
        grid=(A * R // SCW,),
        in_specs=[
            pl.BlockSpec((SCW, 128), lambda i: (i % (T * R // SCW), 0)),
            pl.BlockSpec((1, SCW), lambda i: (0, i)),
        ],
        out_specs=[],
        core_axis_name=("c", "s"),
        dimension_semantics=(pltpu.PARALLEL,),
    )(x_hbm, i_hbm)

  return k(x6, pos6).reshape(P, D)


def _sc_combine_gather(y, pos6):
  y6 = y.reshape(P * R, 128)

  @pl.kernel(out_type=jax.ShapeDtypeStruct((A * R, 128), jnp.float32),
             mesh=_vector_mesh())
  def k(y_hbm, i_hbm, o_hbm):
    def body(i_vmem, o_vmem):
      pltpu.sync_copy(y_hbm.at[i_vmem.at[0]], o_vmem)

    pltpu.emit_pipeline(
        body,
        grid=(A * R // SCW,),
        in_specs=[pl.BlockSpec((1, SCW), lambda i: (0, i))],
        out_specs=[pl.BlockSpec((SCW, 128), lambda i: (i, 0))],
        core_axis_name=("c", "s"),
        dimension_semantics=(pltpu.PARALLEL,),
    )(i_hbm, o_hbm)

  return k(y6, pos6).reshape(A, D)


def _shared_kernel(x_ref, w1_ref, b1_ref, w2_ref, b2_ref, out_ref):
  x = x_ref[...]
  h = jax.nn.gelu(
      jnp.dot(x.astype(jnp.bfloat16), w1_ref[...].astype(jnp.bfloat16),
              preferred_element_type=jnp.float32) + b1_ref[...])
  out_ref[...] = jnp.dot(
      h.astype(jnp.bfloat16), w2_ref[...].astype(jnp.bfloat16),
      preferred_element_type=jnp.float32) + b2_ref[...]


def _shared(x, w1, b1, w2, b2):
  nb = 4
  tb = T // nb
  return pl.pallas_call(
      _shared_kernel,
      grid=(nb,),
      in_specs=[
          pl.BlockSpec((tb, D), lambda i: (i, 0)),
          pl.BlockSpec((D, D), lambda i: (0, 0)),
          pl.BlockSpec((1, D), lambda i: (0, 0)),
          pl.BlockSpec((D, D), lambda i: (0, 0)),
          pl.BlockSpec((1, D), lambda i: (0, 0)),
      ],
      out_specs=pl.BlockSpec((tb, D), lambda i: (i, 0)),
      out_shape=jax.ShapeDtypeStruct((T, D), jnp.float32),
  )(x, w1, b1, w2, b2)


def _grouped_kernel(be_ref, na_ref, xs_ref, w1_ref, b1_ref, w2_ref, b2_ref,
                    y_ref):
  i = pl.program_id(0)

  @pl.when(i < na_ref[0])
  def _():
    x = xs_ref[...]
    h = jax.nn.gelu(
        jnp.dot(x.astype(jnp.bfloat16), w1_ref[0].astype(jnp.bfloat16),
                preferred_element_type=jnp.float32) + b1_ref[0])
    y_ref[...] = jnp.dot(
        h.astype(jnp.bfloat16), w2_ref[0].astype(jnp.bfloat16),
        preferred_element_type=jnp.float32) + b2_ref[0]


def _grouped(xs, we1, be1, we2, be2, block_expert, num_active):
  grid_spec = pltpu.PrefetchScalarGridSpec(
      num_scalar_prefetch=2,
      grid=(G,),
      in_specs=[
          pl.BlockSpec((BLK, D), lambda i, be, na: (i, 0)),
          pl.BlockSpec((1, D, D), lambda i, be, na: (be[i], 0, 0)),
          pl.BlockSpec((1, 1, D), lambda i, be, na: (be[i], 0, 0)),
          pl.BlockSpec((1, D, D), lambda i, be, na: (be[i], 0, 0)),
          pl.BlockSpec((1, 1, D), lambda i, be, na: (be[i], 0, 0)),
      ],
      out_specs=pl.BlockSpec((BLK, D), lambda i, be, na: (i, 0)),
  )
  return pl.pallas_call(
      _grouped_kernel,
      grid_spec=grid_spec,
      out_shape=jax.ShapeDtypeStruct((P, D), jnp.float32),
  )(block_expert, num_active, xs, we1, be1[:, None, :], we2, be2[:, None, :])


def _combine_kernel(sh_ref, g_ref, w0_ref, w1_ref, out_ref):
  i = pl.program_id(0)
  del i
  g0 = g_ref[0]
  g1 = g_ref[1]
  out_ref[...] = sh_ref[...] + w0_ref[...] * g0 + w1_ref[...] * g1


def _combine(sh, g, w0, w1):
  nb = 8
  tb = T // nb
  g3 = g.reshape(K, T, D)
  return pl.pallas_call(
      _combine_kernel,
      grid=(nb,),
      in_specs=[
          pl.BlockSpec((tb, D), lambda i: (i, 0)),
          pl.BlockSpec((K, tb, D), lambda i: (0, i, 0)),
          pl.BlockSpec((tb, 1), lambda i: (i, 0)),
          pl.BlockSpec((tb, 1), lambda i: (i, 0)),
      ],
      out_specs=pl.BlockSpec((tb, D), lambda i: (i, 0)),
      out_shape=jax.ShapeDtypeStruct((T, D), jnp.float32),
  )(sh, g3, w0, w1)


@jax.jit
def kernel(X, Wr, Ws1, bs1, Ws2, bs2, We1, be1, We2, be2):
  x = X[0]
  widx, w, rank, cnt, aux = _router(x, Wr)

  cnt1 = cnt[0]
  padded_blocks = (cnt1 + (BLK - 1)) // BLK
  bstart = jnp.cumsum(padded_blocks) - padded_blocks
  start_rows = bstart * BLK
  pos = jnp.take(start_rows, widx, axis=0) + rank
  pos_flat = jnp.concatenate([pos[:, 0], pos[:, 1]])
  pos6 = (pos_flat[:, None] * R
          + jnp.arange(R, dtype=jnp.int32)[None, :]).reshape(1, A * R)
  gidx = jnp.arange(G, dtype=jnp.int32)
  block_expert = jnp.clip(
      jnp.sum((gidx[:, None] >= bstart[None, :]).astype(jnp.int32),
              axis=1) - 1, 0, E - 1)
  num_active = jnp.sum(padded_blocks, dtype=jnp.int32)[None]

  xs = _sc_dispatch(x, pos6)
  sh = _shared(x, Ws1[0], bs1[0][None, :], Ws2[0], bs2[0][None, :])
  y = _grouped(xs, We1, be1, We2, be2, block_expert, num_active)
  g = _sc_combine_gather(y, pos6)
  out = _combine(sh, g, w[:, 0:1], w[:, 1:2])
  return out[None], aux[0, 0]


# speedup vs baseline: 2.2332x; 2.2332x over previous
import jax
import jax.numpy as jnp
from jax.experimental import pallas as pl
from jax.experimental.pallas import tpu as pltpu

T = 2048
D = 768
E = 8
K = 2
NB = 4
TBT = T // NB


def _fused_kernel(x_ref, wr_ref, ws1_ref, bs1_ref, ws2_ref, bs2_ref,
                  we1_ref, be1_ref, we2_ref, be2_ref, out_ref, aux_ref,
                  acc_ref):
  i = pl.program_id(0)
  x = x_ref[...]

  logits = jnp.dot(x, wr_ref[...], preferred_element_type=jnp.float32)
  m = jnp.max(logits, axis=-1, keepdims=True)
  p = jnp.exp(logits - m)
  p = p / jnp.sum(p, axis=-1, keepdims=True)

  lane = jax.lax.broadcasted_iota(jnp.int32, (TBT, E), 1)
  e0 = jnp.argmax(p, axis=-1).astype(jnp.int32)
  p0 = jnp.max(p, axis=-1)
  oh0 = (lane == e0[:, None]).astype(jnp.float32)
  p_m = jnp.where(oh0 > 0, -jnp.inf, p)
  e1 = jnp.argmax(p_m, axis=-1).astype(jnp.int32)
  p1 = jnp.max(p_m, axis=-1)
  oh1 = (lane == e1[:, None]).astype(jnp.float32)

  s = p0 + p1
  comb = oh0 * (p0 / s)[:, None] + oh1 * (p1 / s)[:, None]

  xb = x.astype(jnp.bfloat16)
  h = jax.nn.gelu(
      jnp.dot(xb, ws1_ref[...], preferred_element_type=jnp.float32)
      + bs1_ref[...])
  out = jnp.dot(h.astype(jnp.bfloat16), ws2_ref[...],
                preferred_element_type=jnp.float32) + bs2_ref[...]

  for e in range(E):
    sc = comb[:, e:e + 1]
    he = jax.nn.gelu(
        jnp.dot(xb, we1_ref[e], preferred_element_type=jnp.float32)
        + be1_ref[e][None, :]) * sc
    out = out + jnp.dot(he.astype(jnp.bfloat16), we2_ref[e],
                        preferred_element_type=jnp.float32)
    out = out + sc * be2_ref[e][None, :]
  out_ref[...] = out

  @pl.when(i == 0)
  def _():
    acc_ref[...] = jnp.zeros_like(acc_ref)

  acc_ref[0:1, :] += jnp.sum(oh0 + oh1, axis=0, keepdims=True)
  acc_ref[1:2, :] += jnp.sum(comb, axis=0, keepdims=True)

  @pl.when(i == NB - 1)
  def _():
    aux_ref[...] = (E / (T * T)) * jnp.sum(
        acc_ref[0:1, :] * acc_ref[1:2, :], axis=-1, keepdims=True)


def _fused(x, wr, ws1b, bs1, ws2b, bs2, we1b, be1, we2b, be2):
  const = lambda i: (0, 0)
  const3 = lambda i: (0, 0, 0)
  return pl.pallas_call(
      _fused_kernel,
      grid=(NB,),
      in_specs=[
          pl.BlockSpec((TBT, D), lambda i: (i, 0)),
          pl.BlockSpec((D, E), const),
          pl.BlockSpec((D, D), const),
          pl.BlockSpec((1, D), const),
          pl.BlockSpec((D, D), const),
          pl.BlockSpec((1, D), const),
          pl.BlockSpec((E, D, D), const3),
          pl.BlockSpec((E, D), const),
          pl.BlockSpec((E, D, D), const3),
          pl.BlockSpec((E, D), const),
      ],
      out_specs=(
          pl.BlockSpec((TBT, D), lambda i: (i, 0)),
          pl.BlockSpec((1, 1), const),
      ),
      out_shape=(
          jax.ShapeDtypeStruct((T, D), jnp.float32),
          jax.ShapeDtypeStruct((1, 1), jnp.float32),
      ),
      scratch_shapes=[pltpu.VMEM((2, E), jnp.float32)],
  )(x, wr, ws1b, bs1, ws2b, bs2, we1b, be1, we2b, be2)


@jax.jit
def kernel(X, Wr, Ws1, bs1, Ws2, bs2, We1, be1, We2, be2):
  x = X[0]
  out, aux = _fused(
      x, Wr,
      Ws1[0].astype(jnp.bfloat16), bs1,
      Ws2[0].astype(jnp.bfloat16), bs2,
      We1.astype(jnp.bfloat16), be1,
      We2.astype(jnp.bfloat16), be2,
  )
  return out[None], aux[0, 0]


# speedup vs baseline: 2.8237x; 1.2644x over previous
import jax
import jax.numpy as jnp
from jax.experimental import pallas as pl
from jax.experimental.pallas import tpu as pltpu

T = 2048
D = 768
E = 8
K = 2
NB = 4
TBT = T // NB


def _fused_kernel(x_ref, wr_ref, ws1_ref, bs1_ref, ws2_ref, bs2_ref,
                  we1_ref, be1_ref, we2_ref, be2_ref, out_ref, aux_ref,
                  acc_ref):
  i = pl.program_id(0)
  x = x_ref[...]

  logits = jnp.dot(x, wr_ref[...], preferred_element_type=jnp.float32)
  m = jnp.max(logits, axis=-1, keepdims=True)
  p = jnp.exp(logits - m)
  p = p / jnp.sum(p, axis=-1, keepdims=True)

  lane = jax.lax.broadcasted_iota(jnp.int32, (TBT, E), 1)
  e0 = jnp.argmax(p, axis=-1).astype(jnp.int32)
  p0 = jnp.max(p, axis=-1)
  oh0 = (lane == e0[:, None]).astype(jnp.float32)
  p_m = jnp.where(oh0 > 0, -jnp.inf, p)
  e1 = jnp.argmax(p_m, axis=-1).astype(jnp.int32)
  p1 = jnp.max(p_m, axis=-1)
  oh1 = (lane == e1[:, None]).astype(jnp.float32)

  s = p0 + p1
  comb = oh0 * (p0 / s)[:, None] + oh1 * (p1 / s)[:, None]

  xb = x.astype(jnp.bfloat16)
  h = jax.nn.gelu(
      (jnp.dot(xb, ws1_ref[...].astype(jnp.bfloat16),
               preferred_element_type=jnp.float32)
       + bs1_ref[...]).astype(jnp.bfloat16))
  out = jnp.dot(h, ws2_ref[...].astype(jnp.bfloat16),
                preferred_element_type=jnp.float32) + bs2_ref[...]

  for e in range(E):
    sc = comb[:, e:e + 1]
    he = jax.nn.gelu(
        (jnp.dot(xb, we1_ref[e].astype(jnp.bfloat16),
                 preferred_element_type=jnp.float32)
         + be1_ref[e][None, :]).astype(jnp.bfloat16)) * sc.astype(jnp.bfloat16)
    out = out + jnp.dot(he, we2_ref[e].astype(jnp.bfloat16),
                        preferred_element_type=jnp.float32)
    out = out + sc * be2_ref[e][None, :]
  out_ref[...] = out

  @pl.when(i == 0)
  def _():
    acc_ref[...] = jnp.zeros_like(acc_ref)

  acc_ref[0:1, :] += jnp.sum(oh0 + oh1, axis=0, keepdims=True)
  acc_ref[1:2, :] += jnp.sum(comb, axis=0, keepdims=True)

  @pl.when(i == NB - 1)
  def _():
    aux_ref[...] = (E / (T * T)) * jnp.sum(
        acc_ref[0:1, :] * acc_ref[1:2, :], axis=-1, keepdims=True)


def _fused(x, wr, ws1b, bs1, ws2b, bs2, we1b, be1, we2b, be2):
  const = lambda i: (0, 0)
  const3 = lambda i: (0, 0, 0)
  return pl.pallas_call(
      _fused_kernel,
      grid=(NB,),
      in_specs=[
          pl.BlockSpec((TBT, D), lambda i: (i, 0)),
          pl.BlockSpec((D, E), const),
          pl.BlockSpec((D, D), const),
          pl.BlockSpec((1, D), const),
          pl.BlockSpec((D, D), const),
          pl.BlockSpec((1, D), const),
          pl.BlockSpec((E, D, D), const3),
          pl.BlockSpec((E, D), const),
          pl.BlockSpec((E, D, D), const3),
          pl.BlockSpec((E, D), const),
      ],
      out_specs=(
          pl.BlockSpec((TBT, D), lambda i: (i, 0)),
          pl.BlockSpec((1, 1), const),
      ),
      out_shape=(
          jax.ShapeDtypeStruct((T, D), jnp.float32),
          jax.ShapeDtypeStruct((1, 1), jnp.float32),
      ),
      scratch_shapes=[pltpu.VMEM((2, E), jnp.float32)],
  )(x, wr, ws1b, bs1, ws2b, bs2, we1b, be1, we2b, be2)


@jax.jit
def kernel(X, Wr, Ws1, bs1, Ws2, bs2, We1, be1, We2, be2):
  x = X[0]
  out, aux = _fused(x, Wr, Ws1[0], bs1, Ws2[0], bs2, We1, be1, We2, be2)
  return out[None], aux[0, 0]
